# baseline (speedup 1.0000x reference)
import functools

import jax
import jax.numpy as jnp
from jax import lax
from jax.experimental import pallas as pl
from jax.experimental.pallas import tpu as pltpu
from jax.experimental.pallas import tpu_sc as plsc

_KH, _KW = 2, 2


def kernel(x, indices):
    B, C, H, W = x.shape
    lut_rank, NK, SS = indices.shape
    oh, ow = H - _KH + 1, W - _KW + 1
    mesh = plsc.VectorSubcoreMesh(core_axis_name="c", subcore_axis_name="s")
    NW = 32
    total = lut_rank * B * NK * SS * oh * ow
    chunk = 15488
    nchunks_per = total // (NW * chunk)

    idxf = indices.reshape(-1).astype(jnp.int32)
    meta = jnp.stack(
        [idxf // (_KH * _KW), (idxf // _KW) % _KH, idxf % _KW] + [idxf * 0] * 13,
        axis=1,
    ).reshape(-1)
    nslots = lut_rank * NK * SS

    @functools.partial(
        pl.kernel,
        mesh=mesh,
        out_type=jax.ShapeDtypeStruct(
            (lut_rank, B, NK, SS, oh, ow), jnp.float32
        ),
        compiler_params=pltpu.CompilerParams(use_tc_tiling_on_sc=False),
        scratch_types=[
            pltpu.VMEM((nslots * 16,), jnp.int32),
            pltpu.VMEM((oh, ow), jnp.float32),
            pltpu.SemaphoreType.DMA,
        ],
    )
    def sc_copy(xd_hbm, meta_hbm, out_hbm, meta_v, buf_v, sem):
        wid = lax.axis_index("s") * 2 + lax.axis_index("c")
        pltpu.sync_copy(meta_hbm, meta_v)
        slots_per = nslots // NW
        nwin = slots_per * B

        def body(i, carry):
            slot = wid * slots_per + i // B
            b = i % B
            mv = meta_v[pl.ds(slot * 16, 16)]
            r = slot // (NK * SS)
            k = (slot // SS) % NK
            s = slot % SS
            pltpu.async_copy(
                xd_hbm.at[mv[2], b, mv[0], pl.ds(mv[1], oh), :], buf_v, sem
            ).wait()
            pltpu.sync_copy(buf_v, out_hbm.at[r, b, k, s])
            return carry

        lax.fori_loop(0, nwin, body, 0)

    xd = jnp.stack([x[..., :ow], x[..., W - ow :]], axis=0)
    out = sc_copy(xd, meta)
    return out.reshape(lut_rank, B, NK, SS, oh * ow)


# speedup vs baseline: 1.2221x; 1.2221x over previous
import functools

import jax
import jax.numpy as jnp
from jax import lax
from jax.experimental import pallas as pl
from jax.experimental.pallas import tpu as pltpu
from jax.experimental.pallas import tpu_sc as plsc

_KH, _KW = 2, 2
_DEPTH = 8


def kernel(x, indices):
    B, C, H, W = x.shape
    lut_rank, NK, SS = indices.shape
    oh, ow = H - _KH + 1, W - _KW + 1
    nslots = lut_rank * NK * SS
    idxf = indices.reshape(-1).astype(jnp.int32)
    meta = jnp.stack(
        [idxf // (_KH * _KW), (idxf // _KW) % _KH, idxf % _KW] + [idxf * 0] * 13,
        axis=1,
    ).reshape(-1)
    xd = jnp.stack([x[..., :ow], x[..., W - ow :]], axis=0)

    mesh = plsc.VectorSubcoreMesh(core_axis_name="c", subcore_axis_name="s")
    NW = 32
    slots_per = nslots // NW
    nwin = slots_per * B

    @functools.partial(
        pl.kernel,
        mesh=mesh,
        out_type=jax.ShapeDtypeStruct((lut_rank, B, NK, SS, oh, ow), jnp.float32),
        compiler_params=pltpu.CompilerParams(use_tc_tiling_on_sc=False),
        scratch_types=[
            pltpu.VMEM((nslots * 16,), jnp.int32),
            pltpu.VMEM((_DEPTH, oh, ow), jnp.float32),
            pltpu.SemaphoreType.DMA,
            pltpu.SemaphoreType.DMA,
        ],
    )
    def sc_copy(xd_hbm, meta_hbm, out_hbm, meta_v, buf_v, sem_in, sem_out):
        wid = lax.axis_index("s") * 2 + lax.axis_index("c")
        pltpu.sync_copy(meta_hbm, meta_v)

        def group_body(g, carry):
            n0 = g * _DEPTH

            def item(j):
                n = n0 + j
                slot = wid * slots_per + n // B
                b = n % B
                mv = meta_v[pl.ds(slot * 16, 16)]
                r = slot // (NK * SS)
                k = (slot // SS) % NK
                s = slot % SS
                return slot, b, mv, r, k, s

            reads = []
            for j in range(_DEPTH):
                _, b, mv, _, _, _ = item(j)
                reads.append(
                    pltpu.async_copy(
                        xd_hbm.at[mv[2], b, mv[0], pl.ds(mv[1], oh), :],
                        buf_v.at[j],
                        sem_in,
                    )
                )
            writes = []
            for j in range(_DEPTH):
                _, b, _, r, k, s = item(j)
                reads[j].wait()
                writes.append(
                    pltpu.async_copy(
                        buf_v.at[j], out_hbm.at[r, b, k, s], sem_out
                    )
                )
            for j in range(_DEPTH):
                writes[j].wait()
            return carry

        lax.fori_loop(0, nwin // _DEPTH, group_body, 0)

    out = sc_copy(xd, meta)
    return out.reshape(lut_rank, B, NK, SS, oh * ow)


# speedup vs baseline: 1.2378x; 1.0129x over previous
import functools

import jax
import jax.numpy as jnp
from jax import lax
from jax.experimental import pallas as pl
from jax.experimental.pallas import tpu as pltpu
from jax.experimental.pallas import tpu_sc as plsc

_KH, _KW = 2, 2
_DEPTH = 8


def kernel(x, indices):
    B, C, H, W = x.shape
    lut_rank, NK, SS = indices.shape
    oh, ow = H - _KH + 1, W - _KW + 1
    nslots = lut_rank * NK * SS
    idxf = indices.reshape(-1).astype(jnp.int32)
    meta = jnp.stack(
        [idxf // (_KH * _KW), (idxf // _KW) % _KH, idxf % _KW] + [idxf * 0] * 13,
        axis=1,
    ).reshape(-1)
    xd = jnp.stack([x[..., :ow], x[..., W - ow :]], axis=0)

    mesh = plsc.VectorSubcoreMesh(core_axis_name="c", subcore_axis_name="s")
    NW = 32
    slots_per = nslots // NW
    NB = 16
    nitems = slots_per * (B // NB)

    @functools.partial(
        pl.kernel,
        mesh=mesh,
        out_type=jax.ShapeDtypeStruct((lut_rank, B, NK, SS, oh, ow), jnp.float32),
        compiler_params=pltpu.CompilerParams(use_tc_tiling_on_sc=False),
        scratch_types=[
            pltpu.VMEM((nslots * 16,), jnp.int32),
            pltpu.VMEM((2, NB, oh, ow), jnp.float32),
            pltpu.SemaphoreType.DMA,
            pltpu.SemaphoreType.DMA,
        ],
    )
    def sc_copy(xd_hbm, meta_hbm, out_hbm, meta_v, buf_v, sem_in, sem_out):
        wid = lax.axis_index("s") * 2 + lax.axis_index("c")
        pltpu.sync_copy(meta_hbm, meta_v)

        def start_read(n):
            slot = wid * slots_per + n // (B // NB)
            b0 = (n % (B // NB)) * NB
            mv = meta_v[pl.ds(slot * 16, 16)]
            return pltpu.async_copy(
                xd_hbm.at[mv[2], pl.ds(b0, NB), mv[0], pl.ds(mv[1], oh), :],
                buf_v.at[n % 2],
                sem_in,
            )

        def start_write(n):
            slot = wid * slots_per + n // (B // NB)
            b0 = (n % (B // NB)) * NB
            r = slot // (NK * SS)
            k = (slot // SS) % NK
            s = slot % SS
            return pltpu.async_copy(
                buf_v.at[n % 2], out_hbm.at[r, pl.ds(b0, NB), k, s], sem_out
            )

        reads = {0: start_read(0)}
        writes = {}
        for n in range(nitems):
            if n >= 1:
                writes[n - 1].wait()
            if n + 1 < nitems:
                reads[n + 1] = start_read(n + 1)
            reads[n].wait()
            writes[n] = start_write(n)
        writes[nitems - 1].wait()

    out = sc_copy(xd, meta)
    return out.reshape(lut_rank, B, NK, SS, oh * ow)
